# baseline (speedup 1.0000x reference)
import functools

import jax
import jax.numpy as jnp
from jax.experimental import pallas as pl
from jax.experimental.pallas import tpu as pltpu
from jax.experimental.pallas import tpu_sc as plsc

DIM = 64
W = 128
WV = 4096
HV = WV // 2


def _tc_transpose(table):
    v = table.shape[0]
    n_blk = pl.cdiv(v, WV)
    t_t = jnp.swapaxes(table, 0, 1)
    mesh = pltpu.create_tensorcore_mesh("core")

    @functools.partial(
        pl.kernel,
        out_type=jax.ShapeDtypeStruct((n_blk * HV, 128), jnp.float32),
        mesh=mesh,
    )
    def transpose_kernel(in_hbm, out_hbm):
        def body(in_vmem, out_vmem):
            tr = jnp.transpose(in_vmem[...])
            out_vmem[:, :DIM] = tr[:HV]
            out_vmem[:, DIM:] = tr[HV:]

        pltpu.emit_pipeline(
            body,
            grid=(n_blk,),
            in_specs=[pl.BlockSpec((DIM, WV), lambda i: (0, i))],
            out_specs=[pl.BlockSpec((HV, 128), lambda i: (i, 0))],
            core_axis_name="core",
            dimension_semantics=(pltpu.PARALLEL,),
        )(in_hbm, out_hbm)

    return transpose_kernel(t_t)


def _sigma(idx):
    i = idx // WV
    r = idx % WV
    return i * WV + jnp.where(r < HV, 2 * r, 2 * (r - HV) + 1)


def _sc_gather(table_lin, idx, n_rows):
    mesh = plsc.VectorSubcoreMesh(core_axis_name="c", subcore_axis_name="s")

    @functools.partial(
        pl.kernel,
        out_type=jax.ShapeDtypeStruct((n_rows, DIM), jnp.float32),
        mesh=mesh,
        compiler_params=pltpu.CompilerParams(use_tc_tiling_on_sc=False),
    )
    def gather_kernel(table_hbm, idx_hbm, rows_hbm):
        def body(i_vmem, o_vmem):
            pltpu.sync_copy(table_hbm.at[i_vmem.at[0]], o_vmem)

        pltpu.emit_pipeline(
            body,
            grid=(n_rows // W,),
            in_specs=[pl.BlockSpec((1, W), index_map=lambda i: (0, i))],
            out_specs=[pl.BlockSpec((W, DIM), index_map=lambda i: (i, 0))],
            core_axis_name=("c", "s"),
            dimension_semantics=(pltpu.PARALLEL,),
        )(idx_hbm, rows_hbm)

    return gather_kernel(table_lin, idx)


def _tc_loss(rows3, vc2):
    k1, half_b, _ = rows3.shape
    blk = 256
    n_blocks = half_b // blk

    def body(rows_ref, vc_ref, out_ref):
        vc_t = jnp.transpose(vc_ref[...])
        ds = []
        for k in range(k1):
            prod_t = jnp.transpose(rows_ref[k]) * vc_t
            d_a = jnp.sum(prod_t[:DIM], axis=0)
            d_b = jnp.sum(prod_t[DIM:], axis=0)
            sgn = 1.0 if k == 0 else -1.0
            ds.append(sgn * d_a)
            ds.append(sgn * d_b)
        dmat = jnp.stack(ds)
        out_ref[0, 0, 0] = jnp.sum(jnp.log(jax.nn.sigmoid(dmat)))

    out = pl.pallas_call(
        body,
        grid=(n_blocks,),
        in_specs=[
            pl.BlockSpec((k1, blk, 128), lambda i: (0, i, 0)),
            pl.BlockSpec((blk, 128), lambda i: (i, 0)),
        ],
        out_specs=pl.BlockSpec(
            (1, 1, 1), lambda i: (i, 0, 0), memory_space=pltpu.SMEM),
        out_shape=jax.ShapeDtypeStruct((n_blocks, 1, 1), jnp.float32),
        compiler_params=pltpu.CompilerParams(
            dimension_semantics=("parallel",)),
    )(rows3, vc2)
    return out


def kernel(centers, contexts, negatives, in_embed, out_embed):
    b = centers.shape[0]
    k1 = 1 + negatives.shape[1]
    v = in_embed.shape[0]
    n_all = k1 * b

    idx_all = jnp.concatenate([contexts[None, :], negatives.T], axis=0)
    idx_all = _sigma(idx_all.reshape(1, -1).astype(jnp.int32))
    idx_c = _sigma(centers[None, :].astype(jnp.int32))

    out_packed = _tc_transpose(out_embed)
    out_lin = out_packed.reshape(out_packed.shape[0] * 2, DIM)
    rows = _sc_gather(out_lin, idx_all, n_all)
    in_packed = _tc_transpose(in_embed)
    in_lin = in_packed.reshape(in_packed.shape[0] * 2, DIM)
    vc = _sc_gather(in_lin, idx_c, b)

    rows3 = rows.reshape(k1, b // 2, 128)
    vc2 = vc.reshape(b // 2, 128)
    partials = _tc_loss(rows3, vc2)
    return -jnp.sum(partials) / b


# speedup vs baseline: 1.3664x; 1.3664x over previous
import functools

import jax
import jax.numpy as jnp
from jax.experimental import pallas as pl
from jax.experimental.pallas import tpu as pltpu
from jax.experimental.pallas import tpu_sc as plsc

DIM = 64
W = 128
WV = 16384
HV = WV // 2


def _tc_transpose(table):
    v = table.shape[0]
    n_blk = pl.cdiv(v, WV)
    t_t = jnp.swapaxes(table, 0, 1)
    mesh = pltpu.create_tensorcore_mesh("core")

    @functools.partial(
        pl.kernel,
        out_type=jax.ShapeDtypeStruct((n_blk * HV, 128), jnp.float32),
        mesh=mesh,
    )
    def transpose_kernel(in_hbm, out_hbm):
        def body(in_vmem, out_vmem):
            tr = jnp.transpose(in_vmem[...])
            out_vmem[:, :DIM] = tr[:HV]
            out_vmem[:, DIM:] = tr[HV:]

        pltpu.emit_pipeline(
            body,
            grid=(n_blk,),
            in_specs=[pl.BlockSpec((DIM, WV), lambda i: (0, i))],
            out_specs=[pl.BlockSpec((HV, 128), lambda i: (i, 0))],
            core_axis_name="core",
            dimension_semantics=(pltpu.PARALLEL,),
        )(in_hbm, out_hbm)

    return transpose_kernel(t_t)


def _sigma(idx):
    i = idx // WV
    r = idx % WV
    return i * WV + jnp.where(r < HV, 2 * r, 2 * (r - HV) + 1)


def _sc_gather(table_lin, idx, n_rows):
    mesh = plsc.VectorSubcoreMesh(core_axis_name="c", subcore_axis_name="s")

    @functools.partial(
        pl.kernel,
        out_type=jax.ShapeDtypeStruct((n_rows, DIM), jnp.float32),
        mesh=mesh,
        compiler_params=pltpu.CompilerParams(use_tc_tiling_on_sc=False),
    )
    def gather_kernel(table_hbm, idx_hbm, rows_hbm):
        def body(i_vmem, o_vmem):
            pltpu.sync_copy(table_hbm.at[i_vmem.at[0]], o_vmem)

        pltpu.emit_pipeline(
            body,
            grid=(n_rows // W,),
            in_specs=[pl.BlockSpec((1, W), index_map=lambda i: (0, i))],
            out_specs=[pl.BlockSpec((W, DIM), index_map=lambda i: (i, 0))],
            core_axis_name=("c", "s"),
            dimension_semantics=(pltpu.PARALLEL,),
        )(idx_hbm, rows_hbm)

    return gather_kernel(table_lin, idx)


def _tc_loss(rows3, vc2):
    k1, half_b, _ = rows3.shape
    blk = 512
    n_blocks = half_b // blk

    def body(rows_ref, vc_ref, out_ref):
        vc_t = jnp.transpose(vc_ref[...])
        ds = []
        for k in range(k1):
            prod_t = jnp.transpose(rows_ref[k]) * vc_t
            d_a = jnp.sum(prod_t[:DIM], axis=0)
            d_b = jnp.sum(prod_t[DIM:], axis=0)
            sgn = 1.0 if k == 0 else -1.0
            ds.append(sgn * d_a)
            ds.append(sgn * d_b)
        dmat = jnp.stack(ds)
        out_ref[0, 0, 0] = jnp.sum(jnp.log(jax.nn.sigmoid(dmat)))

    out = pl.pallas_call(
        body,
        grid=(n_blocks,),
        in_specs=[
            pl.BlockSpec((k1, blk, 128), lambda i: (0, i, 0)),
            pl.BlockSpec((blk, 128), lambda i: (i, 0)),
        ],
        out_specs=pl.BlockSpec(
            (1, 1, 1), lambda i: (i, 0, 0), memory_space=pltpu.SMEM),
        out_shape=jax.ShapeDtypeStruct((n_blocks, 1, 1), jnp.float32),
        compiler_params=pltpu.CompilerParams(
            dimension_semantics=("parallel",)),
    )(rows3, vc2)
    return out


def kernel(centers, contexts, negatives, in_embed, out_embed):
    b = centers.shape[0]
    k1 = 1 + negatives.shape[1]
    v = in_embed.shape[0]
    n_all = k1 * b

    idx_all = jnp.concatenate([contexts[None, :], negatives.T], axis=0)
    idx_all = _sigma(idx_all.reshape(1, -1).astype(jnp.int32))
    idx_c = _sigma(centers[None, :].astype(jnp.int32))

    out_packed = _tc_transpose(out_embed)
    out_lin = out_packed.reshape(out_packed.shape[0] * 2, DIM)
    rows = _sc_gather(out_lin, idx_all, n_all)
    in_packed = _tc_transpose(in_embed)
    in_lin = in_packed.reshape(in_packed.shape[0] * 2, DIM)
    vc = _sc_gather(in_lin, idx_c, b)

    rows3 = rows.reshape(k1, b // 2, 128)
    vc2 = vc.reshape(b // 2, 128)
    partials = _tc_loss(rows3, vc2)
    return -jnp.sum(partials) / b


# speedup vs baseline: 1.4761x; 1.0803x over previous
import functools

import jax
import jax.numpy as jnp
from jax.experimental import pallas as pl
from jax.experimental.pallas import tpu as pltpu
from jax.experimental.pallas import tpu_sc as plsc

DIM = 64
W = 128
WV = 32768
HV = WV // 2
WV_SHIFT = 15


def _tc_transpose(table):
    v = table.shape[0]
    n_blk = pl.cdiv(v, WV)
    t_t = jnp.swapaxes(table, 0, 1)
    mesh = pltpu.create_tensorcore_mesh("core")

    @functools.partial(
        pl.kernel,
        out_type=jax.ShapeDtypeStruct((n_blk * HV, 128), jnp.float32),
        mesh=mesh,
    )
    def transpose_kernel(in_hbm, out_hbm):
        def body(in_vmem, out_vmem):
            tr = jnp.transpose(in_vmem[...])
            out_vmem[:, :DIM] = tr[:HV]
            out_vmem[:, DIM:] = tr[HV:]

        pltpu.emit_pipeline(
            body,
            grid=(n_blk,),
            in_specs=[pl.BlockSpec((DIM, WV), lambda i: (0, i))],
            out_specs=[pl.BlockSpec((HV, 128), lambda i: (i, 0))],
            core_axis_name="core",
            dimension_semantics=(pltpu.PARALLEL,),
        )(in_hbm, out_hbm)

    return transpose_kernel(t_t)


def _sc_gather(table_lin, idx, n_rows):
    mesh = plsc.VectorSubcoreMesh(core_axis_name="c", subcore_axis_name="s")

    @functools.partial(
        pl.kernel,
        out_type=jax.ShapeDtypeStruct((n_rows, DIM), jnp.float32),
        mesh=mesh,
        scratch_types=[pltpu.VMEM((W,), jnp.int32)],
        compiler_params=pltpu.CompilerParams(use_tc_tiling_on_sc=False),
    )
    def gather_kernel(table_hbm, idx_hbm, rows_hbm, sidx):
        def body(i_vmem, o_vmem):
            for c in range(W // 16):
                v = i_vmem[0, pl.ds(c * 16, 16)]
                blk = (v >> WV_SHIFT) << WV_SHIFT
                r = v - blk
                sig = blk + jnp.where(r < HV, 2 * r, 2 * (r - HV) + 1)
                sidx[pl.ds(c * 16, 16)] = sig
            pltpu.sync_copy(table_hbm.at[sidx], o_vmem)

        pltpu.emit_pipeline(
            body,
            grid=(n_rows // W,),
            in_specs=[pl.BlockSpec((1, W), index_map=lambda i: (0, i))],
            out_specs=[pl.BlockSpec((W, DIM), index_map=lambda i: (i, 0))],
            core_axis_name=("c", "s"),
            dimension_semantics=(pltpu.PARALLEL,),
        )(idx_hbm, rows_hbm)

    return gather_kernel(table_lin, idx)


def _tc_loss(rows3, vc2):
    k1, half_b, _ = rows3.shape
    blk = 512
    n_blocks = half_b // blk

    def body(rows_ref, vc_ref, out_ref):
        vc_t = jnp.transpose(vc_ref[...])
        ds = []
        for k in range(k1):
            prod_t = jnp.transpose(rows_ref[k]) * vc_t
            d_a = jnp.sum(prod_t[:DIM], axis=0)
            d_b = jnp.sum(prod_t[DIM:], axis=0)
            sgn = 1.0 if k == 0 else -1.0
            ds.append(sgn * d_a)
            ds.append(sgn * d_b)
        dmat = jnp.stack(ds)
        out_ref[0, 0, 0] = jnp.sum(jnp.log(jax.nn.sigmoid(dmat)))

    out = pl.pallas_call(
        body,
        grid=(n_blocks,),
        in_specs=[
            pl.BlockSpec((k1, blk, 128), lambda i: (0, i, 0)),
            pl.BlockSpec((blk, 128), lambda i: (i, 0)),
        ],
        out_specs=pl.BlockSpec(
            (1, 1, 1), lambda i: (i, 0, 0), memory_space=pltpu.SMEM),
        out_shape=jax.ShapeDtypeStruct((n_blocks, 1, 1), jnp.float32),
        compiler_params=pltpu.CompilerParams(
            dimension_semantics=("parallel",)),
    )(rows3, vc2)
    return out


def kernel(centers, contexts, negatives, in_embed, out_embed):
    b = centers.shape[0]
    k1 = 1 + negatives.shape[1]
    v = in_embed.shape[0]
    n_all = k1 * b

    idx_all = jnp.concatenate([contexts[None, :], negatives.T], axis=0)
    idx_all = idx_all.reshape(1, -1).astype(jnp.int32)
    idx_c = centers[None, :].astype(jnp.int32)

    out_packed = _tc_transpose(out_embed)
    out_lin = out_packed.reshape(out_packed.shape[0] * 2, DIM)
    rows = _sc_gather(out_lin, idx_all, n_all)
    in_packed = _tc_transpose(in_embed)
    in_lin = in_packed.reshape(in_packed.shape[0] * 2, DIM)
    vc = _sc_gather(in_lin, idx_c, b)

    rows3 = rows.reshape(k1, b // 2, 128)
    vc2 = vc.reshape(b // 2, 128)
    partials = _tc_loss(rows3, vc2)
    return -jnp.sum(partials) / b
